# Initial kernel scaffold; baseline (speedup 1.0000x reference)
import jax
import jax.numpy as jnp
from jax.experimental import pallas as pl


def kernel(x, edge_index, batch, W0, b0, W1, b1, Wfc, bfc, W2, b2, W3, b3):
    raise NotImplementedError("write your pallas kernel here")


# speedup vs baseline: 9.5280x; 9.5280x over previous
import functools
import jax
import jax.numpy as jnp
from jax import lax
from jax.experimental import pallas as pl
from jax.experimental.pallas import tpu as pltpu
import jax.experimental.pallas.tpu_sc as plsc

N = 10000
E = 320000
F = 128
H = 256
B = 8

NS = 16
NC = 2
EPT = E // NS
K = 128
NBP = -(-EPT // K)
EPTP = NBP * K
ACC_R = N + 8
CH = 632
CH_LAST = N - (NS - 1) * CH


def _per_tile_rows(do_copy):
    t = lax.axis_index("s")

    @pl.when(t < NS - 1)
    def _():
        do_copy(pl.multiple_of(t * CH, 8), CH)

    @pl.when(t == NS - 1)
    def _():
        do_copy((NS - 1) * CH, CH_LAST)

@functools.lru_cache(maxsize=None)
def _sc_mesh():
    return plsc.VectorSubcoreMesh(
        core_axis_name="c", subcore_axis_name="s",
        num_cores=NC, num_subcores=NS)


def _deg_body(dst_hbm, zeros_hbm, ones_hbm, out_hbm,
              acc, ones_v, didx0, didx1, semd0, semd1):
    c = lax.axis_index("c")
    t = lax.axis_index("s")
    _per_tile_rows(lambda b, n: pltpu.sync_copy(
        zeros_hbm.at[pl.ds(b, n)], acc.at[pl.ds(b, n)]))
    pltpu.sync_copy(ones_hbm, ones_v)
    plsc.subcore_barrier()

    def dslc(gb):
        return dst_hbm.at[pl.ds(pl.multiple_of(t * EPTP + gb * K, 8), K)]

    pltpu.async_copy(dslc(0), didx0, semd0)
    pltpu.async_copy(dslc(1), didx1, semd1)

    def step(gb, didx, semd):
        pltpu.make_async_copy(dslc(gb), didx, semd).wait()
        pltpu.sync_copy(ones_v, acc.at[didx], add=True)

        @pl.when(gb + 2 < NBP)
        def _():
            pltpu.async_copy(dslc(gb + 2), didx, semd)

    @pl.loop(0, NBP - 1, step=2)
    def _(g):
        step(g, didx0, semd0)
        step(g + 1, didx1, semd1)

    step(NBP - 1, didx0, semd0)

    plsc.subcore_barrier()
    _per_tile_rows(lambda b, n: pltpu.sync_copy(
        acc.at[pl.ds(b, n)], out_hbm.at[c, pl.ds(b, n)]))


@jax.jit
def _deg_sc(dst_r, zeros16, ones_k):
    f = pl.kernel(
        _deg_body,
        out_type=jax.ShapeDtypeStruct((NC, N, 128), jnp.float32),
        mesh=_sc_mesh(),
        scratch_types=[
            pltpu.VMEM_SHARED((ACC_R, 128), jnp.float32),
            pltpu.VMEM((K, 128), jnp.float32),
            pltpu.VMEM((K,), jnp.int32),
            pltpu.VMEM((K,), jnp.int32),
            pltpu.SemaphoreType.DMA,
            pltpu.SemaphoreType.DMA,
        ],
    )
    return f(dst_r, zeros16, ones_k)


def _spmm_body(D, table_hbm, src_hbm, dst_hbm, out_hbm,
               acc, sidx0, sidx1, didx0, didx1, rows0, rows1,
               sem0, sem1, semd0, semd1):
    c = lax.axis_index("c")
    t = lax.axis_index("s")
    _per_tile_rows(lambda b, n: pltpu.sync_copy(
        table_hbm.at[pl.ds(pl.multiple_of(c * N + b, 8), n)],
        acc.at[pl.ds(b, n)]))
    plsc.subcore_barrier()

    bufs = ((sidx0, didx0, rows0, sem0, semd0),
            (sidx1, didx1, rows1, sem1, semd1))

    def sslc(gb):
        return src_hbm.at[
            pl.ds(pl.multiple_of((c * NS + t) * EPTP + gb * K, 8), K)]

    def dslc(gb):
        return dst_hbm.at[pl.ds(pl.multiple_of(t * EPTP + gb * K, 8), K)]

    for gb in (0, 1):
        sidx, _, rows, sem, _ = bufs[gb]
        pltpu.sync_copy(sslc(gb), sidx)
        pltpu.async_copy(table_hbm.at[sidx], rows, sem)
    pltpu.async_copy(dslc(0), didx0, semd0)

    def step(gb, b):
        sidx, didx, rows, sem, semd = bufs[b]
        pltpu.make_async_copy(dslc(gb), didx, semd).wait()

        @pl.when(gb + 1 < NBP)
        def _():
            ob = bufs[1 - b]
            pltpu.async_copy(dslc(gb + 1), ob[1], ob[4])

        pltpu.make_async_copy(table_hbm.at[sidx], rows, sem).wait()
        pltpu.sync_copy(rows, acc.at[didx], add=True)

        @pl.when(gb + 2 < NBP)
        def _():
            pltpu.sync_copy(sslc(gb + 2), sidx)
            pltpu.async_copy(table_hbm.at[sidx], rows, sem)

    @pl.loop(0, NBP - 1, step=2)
    def _(g):
        step(g, 0)
        step(g + 1, 1)

    step(NBP - 1, 0)

    plsc.subcore_barrier()
    _per_tile_rows(lambda b, n: pltpu.sync_copy(
        acc.at[pl.ds(b, n)], out_hbm.at[c, pl.ds(b, n)]))


@functools.partial(jax.jit, static_argnums=(3,))
def _spmm_sc(table2, src2, dst_r, D):
    f = pl.kernel(
        functools.partial(_spmm_body, D),
        out_type=jax.ShapeDtypeStruct((NC, N, D), jnp.float32),
        mesh=_sc_mesh(),
        scratch_types=[
            pltpu.VMEM_SHARED((ACC_R, D), jnp.float32),
            pltpu.VMEM((K,), jnp.int32),
            pltpu.VMEM((K,), jnp.int32),
            pltpu.VMEM((K,), jnp.int32),
            pltpu.VMEM((K,), jnp.int32),
            pltpu.VMEM((K, D), jnp.float32),
            pltpu.VMEM((K, D), jnp.float32),
            pltpu.SemaphoreType.DMA,
            pltpu.SemaphoreType.DMA,
            pltpu.SemaphoreType.DMA,
            pltpu.SemaphoreType.DMA,
        ],
    )
    return f(table2, src2, dst_r)


MB = 1000
NMB = N // MB


def _t1_body(x_ref, w_ref, dcol_ref, hws_ref, dinv_ref):
    dinv = lax.rsqrt(dcol_ref[...] + 1.0)
    dinv_ref[...] = dinv
    hws_ref[0] = dinv * jnp.dot(x_ref[...], w_ref[...],
                                preferred_element_type=jnp.float32)


@jax.jit
def _t1(x, W0, dcol):
    return pl.pallas_call(
        _t1_body,
        grid=(NC, NMB),
        in_specs=[
            pl.BlockSpec((MB, F), lambda c, m: (m, 0)),
            pl.BlockSpec((F, H // NC), lambda c, m: (0, c)),
            pl.BlockSpec((MB, 1), lambda c, m: (m, 0)),
        ],
        out_specs=[
            pl.BlockSpec((1, MB, H // NC), lambda c, m: (c, m, 0)),
            pl.BlockSpec((MB, 1), lambda c, m: (m, 0)),
        ],
        out_shape=[
            jax.ShapeDtypeStruct((NC, N, H // NC), jnp.float32),
            jax.ShapeDtypeStruct((N, 1), jnp.float32),
        ],
    )(x, W0, dcol)


def _epi_mm_body(with_glob, raw_ref, dinv_ref, b_ref, w_ref, oh_ref,
                 hws_ref, glob_ref):
    c = pl.program_id(0)
    mb = pl.program_id(1)
    dinv = dinv_ref[...]
    h = jax.nn.relu(
        dinv * jnp.concatenate([raw_ref[0], raw_ref[1]], axis=-1) + b_ref[...])
    hws_ref[0] = dinv * jnp.dot(h, w_ref[...],
                                preferred_element_type=jnp.float32)
    if with_glob:
        hh = jnp.where(c == 0, h[:, :H // NC], h[:, H // NC:])
        oh = oh_ref[...]
        neg = jnp.float32(-jnp.inf)
        parts = [jnp.max(jnp.where(oh[:, g:g + 1] > 0, hh, neg), axis=0)
                 for g in range(B)]
        blk = jnp.stack(parts, axis=0)
        prev = jnp.where(mb == 0, neg, glob_ref[...])
        glob_ref[...] = jnp.maximum(prev, blk)


@jax.jit
def _t2(raw0, dinv, b0, W1, onehot):
    return pl.pallas_call(
        functools.partial(_epi_mm_body, True),
        grid=(NC, NMB),
        in_specs=[
            pl.BlockSpec((NC, MB, H // NC), lambda c, m: (0, m, 0)),
            pl.BlockSpec((MB, 1), lambda c, m: (m, 0)),
            pl.BlockSpec((1, H), lambda c, m: (0, 0)),
            pl.BlockSpec((H, H // NC), lambda c, m: (0, c)),
            pl.BlockSpec((MB, B), lambda c, m: (m, 0)),
        ],
        out_specs=[
            pl.BlockSpec((1, MB, H // NC), lambda c, m: (c, m, 0)),
            pl.BlockSpec((B, H // NC), lambda c, m: (0, c)),
        ],
        out_shape=[
            jax.ShapeDtypeStruct((NC, N, H // NC), jnp.float32),
            jax.ShapeDtypeStruct((B, H), jnp.float32),
        ],
    )(raw0, dinv, b0, W1, onehot)


def _t4_body(raw_ref, dinv_ref, b_ref, w_ref, hws_ref):
    _epi_mm_body(False, raw_ref, dinv_ref, b_ref, w_ref, None, hws_ref, None)


@jax.jit
def _t4(raw, dinv, b, W):
    return pl.pallas_call(
        _t4_body,
        grid=(NC, NMB),
        in_specs=[
            pl.BlockSpec((NC, MB, H // NC), lambda c, m: (0, m, 0)),
            pl.BlockSpec((MB, 1), lambda c, m: (m, 0)),
            pl.BlockSpec((1, H), lambda c, m: (0, 0)),
            pl.BlockSpec((H, H // NC), lambda c, m: (0, c)),
        ],
        out_specs=pl.BlockSpec((1, MB, H // NC), lambda c, m: (c, m, 0)),
        out_shape=jax.ShapeDtypeStruct((NC, N, H // NC), jnp.float32),
    )(raw, dinv, b, W)


def _t3_body(glob_ref, wfc_ref, bfc_ref, w2b_ref, g2_ref):
    gi = jnp.dot(glob_ref[...], wfc_ref[...],
                 preferred_element_type=jnp.float32) + bfc_ref[...]
    g2_ref[...] = jnp.dot(gi, w2b_ref[...],
                          preferred_element_type=jnp.float32)


@jax.jit
def _t3(glob, Wfc, bfc, W2b):
    return pl.pallas_call(
        _t3_body,
        out_shape=jax.ShapeDtypeStruct((B, H), jnp.float32),
    )(glob, Wfc, bfc, W2b)


def _t5_body(raw_ref, dinv_ref, b_ref, w_ref, g2_ref, oh_ref, hws_ref):
    dinv = dinv_ref[...]
    h = jax.nn.relu(
        dinv * jnp.concatenate([raw_ref[0], raw_ref[1]], axis=-1) + b_ref[...])
    mm = jnp.dot(h, w_ref[...], preferred_element_type=jnp.float32)
    gb = jnp.dot(oh_ref[...], g2_ref[...], preferred_element_type=jnp.float32)
    hws_ref[0] = dinv * (mm + gb)


@jax.jit
def _t5(raw, dinv, b1, W2t, G2, onehot):
    return pl.pallas_call(
        _t5_body,
        grid=(NC, NMB),
        in_specs=[
            pl.BlockSpec((NC, MB, H // NC), lambda c, m: (0, m, 0)),
            pl.BlockSpec((MB, 1), lambda c, m: (m, 0)),
            pl.BlockSpec((1, H), lambda c, m: (0, 0)),
            pl.BlockSpec((H, H // NC), lambda c, m: (0, c)),
            pl.BlockSpec((B, H // NC), lambda c, m: (0, c)),
            pl.BlockSpec((MB, B), lambda c, m: (m, 0)),
        ],
        out_specs=pl.BlockSpec((1, MB, H // NC), lambda c, m: (c, m, 0)),
        out_shape=jax.ShapeDtypeStruct((NC, N, H // NC), jnp.float32),
    )(raw, dinv, b1, W2t, G2, onehot)


def _t6_body(raw_ref, dinv_ref, b_ref, w_ref, hws_ref):
    dinv = dinv_ref[...]
    h = jax.nn.relu(
        dinv * jnp.concatenate([raw_ref[0], raw_ref[1]], axis=-1) + b_ref[...])
    hws_ref[0] = dinv * jnp.dot(h, w_ref[...],
                                preferred_element_type=jnp.float32)


@jax.jit
def _t6(raw, dinv, b2, W3p):
    return pl.pallas_call(
        _t6_body,
        grid=(NC, NMB),
        in_specs=[
            pl.BlockSpec((NC, MB, H // NC), lambda c, m: (0, m, 0)),
            pl.BlockSpec((MB, 1), lambda c, m: (m, 0)),
            pl.BlockSpec((1, H), lambda c, m: (0, 0)),
            pl.BlockSpec((H, H // NC), lambda c, m: (0, 0)),
        ],
        out_specs=pl.BlockSpec((1, MB, H // NC), lambda c, m: (c, m, 0)),
        out_shape=jax.ShapeDtypeStruct((NC, N, H // NC), jnp.float32),
    )(raw, dinv, b2, W3p)


def _t7_body(raw_ref, dinv_ref, b3_ref, out_ref):
    v = dinv_ref[...] * raw_ref[0, :, 0:1] + b3_ref[...]
    out_ref[...] = v


@jax.jit
def _t7(raw3, dinv, b3):
    return pl.pallas_call(
        _t7_body,
        grid=(NMB,),
        in_specs=[
            pl.BlockSpec((1, MB, H // NC), lambda m: (0, m, 0)),
            pl.BlockSpec((MB, 1), lambda m: (m, 0)),
            pl.BlockSpec((1, 1), lambda m: (0, 0)),
        ],
        out_specs=pl.BlockSpec((MB, 1), lambda m: (m, 0)),
        out_shape=jax.ShapeDtypeStruct((N, 1), jnp.float32),
    )(raw3, dinv, b3)


def kernel(x, edge_index, batch, W0, b0, W1, b1, Wfc, bfc, W2, b2, W3, b3):
    src = edge_index[0]
    dst = edge_index[1]
    srcp = jnp.pad(src.reshape(NS, EPT), ((0, 0), (0, EPTP - EPT)))
    src2 = jnp.stack([srcp, srcp + N]).reshape(-1)
    dst_r = jnp.pad(dst.reshape(NS, EPT), ((0, 0), (0, EPTP - EPT)),
                    constant_values=N).reshape(-1)
    onehot = (batch[:, None] == jnp.arange(B)[None, :]).astype(jnp.float32)
    zeros16 = jnp.zeros((N, 128), jnp.float32)
    ones_k = jnp.ones((K, 128), jnp.float32)
    W3p = jnp.pad(W3, ((0, 0), (0, H // NC - 1)))
    b0r = b0.reshape(1, H)
    b1r = b1.reshape(1, H)
    b2r = b2.reshape(1, H)
    bfr = bfc.reshape(1, H)
    b3r = b3.reshape(1, 1)
    W2t = W2[:H]
    W2b = W2[H:]

    deg_raw = _deg_sc(dst_r, zeros16, ones_k)
    dcol = deg_raw[0, :, 0:1]

    hws0, dinv = _t1(x, W0, dcol)
    raw0 = _spmm_sc(hws0.reshape(NC * N, H // NC), src2, dst_r, H // NC)
    hws1, glob = _t2(raw0, dinv, b0r, W1, onehot)
    g2 = _t3(glob, Wfc, bfr, W2b)
    raw1 = _spmm_sc(hws1.reshape(NC * N, H // NC), src2, dst_r, H // NC)
    hws1b = _t4(raw1, dinv, b1r, W1)
    raw1b = _spmm_sc(hws1b.reshape(NC * N, H // NC), src2, dst_r, H // NC)
    hws2 = _t5(raw1b, dinv, b1r, W2t, g2, onehot)
    raw2 = _spmm_sc(hws2.reshape(NC * N, H // NC), src2, dst_r, H // NC)
    hws3 = _t6(raw2, dinv, b2r, W3p)
    raw3 = _spmm_sc(hws3.reshape(NC * N, H // NC), src2, dst_r, H // NC)
    return _t7(raw3, dinv, b3r)


# speedup vs baseline: 10.5513x; 1.1074x over previous
import functools
import jax
import jax.numpy as jnp
from jax import lax
from jax.experimental import pallas as pl
from jax.experimental.pallas import tpu as pltpu
import jax.experimental.pallas.tpu_sc as plsc

N = 10000
E = 320000
F = 128
H = 256
B = 8

NS = 16
NC = 2
EPT = E // NS
K = 128
NBP = -(-EPT // K)
EPTP = NBP * K
ACC_R = N + 8
CH = 632
CH_LAST = N - (NS - 1) * CH


def _per_tile_rows(do_copy):
    t = lax.axis_index("s")

    @pl.when(t < NS - 1)
    def _():
        do_copy(pl.multiple_of(t * CH, 8), CH)

    @pl.when(t == NS - 1)
    def _():
        do_copy((NS - 1) * CH, CH_LAST)

@functools.lru_cache(maxsize=None)
def _sc_mesh():
    return plsc.VectorSubcoreMesh(
        core_axis_name="c", subcore_axis_name="s",
        num_cores=NC, num_subcores=NS)


def _deg_body(dst_hbm, zeros_hbm, ones_hbm, out_hbm,
              acc, ones_v, didx0, didx1, didx2,
              semd0, semd1, semd2, sems0, sems1, sems2):
    c = lax.axis_index("c")
    t = lax.axis_index("s")
    _per_tile_rows(lambda b, n: pltpu.sync_copy(
        zeros_hbm.at[pl.ds(b, n)], acc.at[pl.ds(b, n)]))
    pltpu.sync_copy(ones_hbm, ones_v)
    plsc.subcore_barrier()

    bufs = ((didx0, semd0, sems0), (didx1, semd1, sems1),
            (didx2, semd2, sems2))

    def dslc(gb):
        return dst_hbm.at[pl.ds(pl.multiple_of(t * EPTP + gb * K, 8), K)]

    def prep(nb, pb):
        didx, semd, sems = bufs[pb]

        @pl.when(nb >= 3)
        def _():
            pltpu.make_async_copy(ones_v, acc.at[didx], sems).wait()

        pltpu.async_copy(dslc(nb), didx, semd)

    def proc(gb, b):
        didx, semd, sems = bufs[b]
        pltpu.make_async_copy(dslc(gb), didx, semd).wait()
        pltpu.async_copy(ones_v, acc.at[didx], sems, add=True)

    prep(0, 0)
    prep(1, 1)

    @pl.loop(0, NBP - 1, step=3)
    def _(g):
        for i in range(3):
            gb = g + i
            proc(gb, i)

            @pl.when(gb + 2 < NBP)
            def _():
                prep(gb + 2, (i + 2) % 3)

    proc(NBP - 1, (NBP - 1) % 3)
    for gb in (NBP - 3, NBP - 2, NBP - 1):
        didx, _, sems = bufs[gb % 3]
        pltpu.make_async_copy(ones_v, acc.at[didx], sems).wait()

    plsc.subcore_barrier()
    _per_tile_rows(lambda b, n: pltpu.sync_copy(
        acc.at[pl.ds(b, n)], out_hbm.at[c, pl.ds(b, n)]))


@jax.jit
def _deg_sc(dst_r, zeros16, ones_k):
    f = pl.kernel(
        _deg_body,
        out_type=jax.ShapeDtypeStruct((NC, N, 128), jnp.float32),
        mesh=_sc_mesh(),
        scratch_types=(
            [pltpu.VMEM_SHARED((ACC_R, 128), jnp.float32),
             pltpu.VMEM((K, 128), jnp.float32)]
            + [pltpu.VMEM((K,), jnp.int32)] * 3
            + [pltpu.SemaphoreType.DMA] * 6
        ),
    )
    return f(dst_r, zeros16, ones_k)


def _spmm_body(D, table_hbm, src_hbm, dst_hbm, out_hbm,
               acc, sidx0, sidx1, sidx2, didx0, didx1, didx2,
               rows0, rows1, rows2,
               semg0, semg1, semg2, semd0, semd1, semd2,
               sems0, sems1, sems2):
    c = lax.axis_index("c")
    t = lax.axis_index("s")
    _per_tile_rows(lambda b, n: pltpu.sync_copy(
        table_hbm.at[pl.ds(pl.multiple_of(c * N + b, 8), n)],
        acc.at[pl.ds(b, n)]))
    plsc.subcore_barrier()

    bufs = ((sidx0, didx0, rows0, semg0, semd0, sems0),
            (sidx1, didx1, rows1, semg1, semd1, sems1),
            (sidx2, didx2, rows2, semg2, semd2, sems2))

    def sslc(gb):
        return src_hbm.at[
            pl.ds(pl.multiple_of((c * NS + t) * EPTP + gb * K, 8), K)]

    def dslc(gb):
        return dst_hbm.at[pl.ds(pl.multiple_of(t * EPTP + gb * K, 8), K)]

    def prep(nb, pb):
        sidx, didx, rows, semg, semd, sems = bufs[pb]

        @pl.when(nb >= 3)
        def _():
            pltpu.make_async_copy(rows, acc.at[didx], sems).wait()

        pltpu.sync_copy(sslc(nb), sidx)
        pltpu.async_copy(table_hbm.at[sidx], rows, semg)
        pltpu.async_copy(dslc(nb), didx, semd)

    def proc(gb, b):
        sidx, didx, rows, semg, semd, sems = bufs[b]
        pltpu.make_async_copy(dslc(gb), didx, semd).wait()
        pltpu.make_async_copy(table_hbm.at[sidx], rows, semg).wait()
        pltpu.async_copy(rows, acc.at[didx], sems, add=True)

    prep(0, 0)
    prep(1, 1)

    @pl.loop(0, NBP - 1, step=3)
    def _(g):
        for i in range(3):
            gb = g + i
            proc(gb, i)

            @pl.when(gb + 2 < NBP)
            def _():
                prep(gb + 2, (i + 2) % 3)

    proc(NBP - 1, (NBP - 1) % 3)

    for gb in (NBP - 3, NBP - 2, NBP - 1):
        _, didx, rows, _, _, sems = bufs[gb % 3]
        pltpu.make_async_copy(rows, acc.at[didx], sems).wait()

    plsc.subcore_barrier()
    _per_tile_rows(lambda b, n: pltpu.sync_copy(
        acc.at[pl.ds(b, n)], out_hbm.at[c, pl.ds(b, n)]))


@functools.partial(jax.jit, static_argnums=(3,))
def _spmm_sc(table2, src2, dst_r, D):
    f = pl.kernel(
        functools.partial(_spmm_body, D),
        out_type=jax.ShapeDtypeStruct((NC, N, D), jnp.float32),
        mesh=_sc_mesh(),
        scratch_types=(
            [pltpu.VMEM_SHARED((ACC_R, D), jnp.float32)]
            + [pltpu.VMEM((K,), jnp.int32)] * 6
            + [pltpu.VMEM((K, D), jnp.float32)] * 3
            + [pltpu.SemaphoreType.DMA] * 9
        ),
    )
    return f(table2, src2, dst_r)


MB = 1000
NMB = N // MB


def _t1_body(x_ref, w_ref, dcol_ref, hws_ref, dinv_ref):
    dinv = lax.rsqrt(dcol_ref[...] + 1.0)
    dinv_ref[...] = dinv
    hws_ref[0] = dinv * jnp.dot(x_ref[...], w_ref[...],
                                preferred_element_type=jnp.float32)


@jax.jit
def _t1(x, W0, dcol):
    return pl.pallas_call(
        _t1_body,
        grid=(NC, NMB),
        in_specs=[
            pl.BlockSpec((MB, F), lambda c, m: (m, 0)),
            pl.BlockSpec((F, H // NC), lambda c, m: (0, c)),
            pl.BlockSpec((MB, 1), lambda c, m: (m, 0)),
        ],
        out_specs=[
            pl.BlockSpec((1, MB, H // NC), lambda c, m: (c, m, 0)),
            pl.BlockSpec((MB, 1), lambda c, m: (m, 0)),
        ],
        out_shape=[
            jax.ShapeDtypeStruct((NC, N, H // NC), jnp.float32),
            jax.ShapeDtypeStruct((N, 1), jnp.float32),
        ],
    )(x, W0, dcol)


def _epi_mm_body(with_glob, raw_ref, dinv_ref, b_ref, w_ref, oh_ref,
                 hws_ref, glob_ref):
    c = pl.program_id(0)
    mb = pl.program_id(1)
    dinv = dinv_ref[...]
    h = jax.nn.relu(
        dinv * jnp.concatenate([raw_ref[0], raw_ref[1]], axis=-1) + b_ref[...])
    hws_ref[0] = dinv * jnp.dot(h, w_ref[...],
                                preferred_element_type=jnp.float32)
    if with_glob:
        hh = jnp.where(c == 0, h[:, :H // NC], h[:, H // NC:])
        oh = oh_ref[...]
        neg = jnp.float32(-jnp.inf)
        parts = [jnp.max(jnp.where(oh[:, g:g + 1] > 0, hh, neg), axis=0)
                 for g in range(B)]
        blk = jnp.stack(parts, axis=0)
        prev = jnp.where(mb == 0, neg, glob_ref[...])
        glob_ref[...] = jnp.maximum(prev, blk)


@jax.jit
def _t2(raw0, dinv, b0, W1, onehot):
    return pl.pallas_call(
        functools.partial(_epi_mm_body, True),
        grid=(NC, NMB),
        in_specs=[
            pl.BlockSpec((NC, MB, H // NC), lambda c, m: (0, m, 0)),
            pl.BlockSpec((MB, 1), lambda c, m: (m, 0)),
            pl.BlockSpec((1, H), lambda c, m: (0, 0)),
            pl.BlockSpec((H, H // NC), lambda c, m: (0, c)),
            pl.BlockSpec((MB, B), lambda c, m: (m, 0)),
        ],
        out_specs=[
            pl.BlockSpec((1, MB, H // NC), lambda c, m: (c, m, 0)),
            pl.BlockSpec((B, H // NC), lambda c, m: (0, c)),
        ],
        out_shape=[
            jax.ShapeDtypeStruct((NC, N, H // NC), jnp.float32),
            jax.ShapeDtypeStruct((B, H), jnp.float32),
        ],
    )(raw0, dinv, b0, W1, onehot)


def _t4_body(raw_ref, dinv_ref, b_ref, w_ref, hws_ref):
    _epi_mm_body(False, raw_ref, dinv_ref, b_ref, w_ref, None, hws_ref, None)


@jax.jit
def _t4(raw, dinv, b, W):
    return pl.pallas_call(
        _t4_body,
        grid=(NC, NMB),
        in_specs=[
            pl.BlockSpec((NC, MB, H // NC), lambda c, m: (0, m, 0)),
            pl.BlockSpec((MB, 1), lambda c, m: (m, 0)),
            pl.BlockSpec((1, H), lambda c, m: (0, 0)),
            pl.BlockSpec((H, H // NC), lambda c, m: (0, c)),
        ],
        out_specs=pl.BlockSpec((1, MB, H // NC), lambda c, m: (c, m, 0)),
        out_shape=jax.ShapeDtypeStruct((NC, N, H // NC), jnp.float32),
    )(raw, dinv, b, W)


def _t3_body(glob_ref, wfc_ref, bfc_ref, w2b_ref, g2_ref):
    gi = jnp.dot(glob_ref[...], wfc_ref[...],
                 preferred_element_type=jnp.float32) + bfc_ref[...]
    g2_ref[...] = jnp.dot(gi, w2b_ref[...],
                          preferred_element_type=jnp.float32)


@jax.jit
def _t3(glob, Wfc, bfc, W2b):
    return pl.pallas_call(
        _t3_body,
        out_shape=jax.ShapeDtypeStruct((B, H), jnp.float32),
    )(glob, Wfc, bfc, W2b)


def _t5_body(raw_ref, dinv_ref, b_ref, w_ref, g2_ref, oh_ref, hws_ref):
    dinv = dinv_ref[...]
    h = jax.nn.relu(
        dinv * jnp.concatenate([raw_ref[0], raw_ref[1]], axis=-1) + b_ref[...])
    mm = jnp.dot(h, w_ref[...], preferred_element_type=jnp.float32)
    gb = jnp.dot(oh_ref[...], g2_ref[...], preferred_element_type=jnp.float32)
    hws_ref[0] = dinv * (mm + gb)


@jax.jit
def _t5(raw, dinv, b1, W2t, G2, onehot):
    return pl.pallas_call(
        _t5_body,
        grid=(NC, NMB),
        in_specs=[
            pl.BlockSpec((NC, MB, H // NC), lambda c, m: (0, m, 0)),
            pl.BlockSpec((MB, 1), lambda c, m: (m, 0)),
            pl.BlockSpec((1, H), lambda c, m: (0, 0)),
            pl.BlockSpec((H, H // NC), lambda c, m: (0, c)),
            pl.BlockSpec((B, H // NC), lambda c, m: (0, c)),
            pl.BlockSpec((MB, B), lambda c, m: (m, 0)),
        ],
        out_specs=pl.BlockSpec((1, MB, H // NC), lambda c, m: (c, m, 0)),
        out_shape=jax.ShapeDtypeStruct((NC, N, H // NC), jnp.float32),
    )(raw, dinv, b1, W2t, G2, onehot)


def _t6_body(raw_ref, dinv_ref, b_ref, w_ref, hws_ref):
    dinv = dinv_ref[...]
    h = jax.nn.relu(
        dinv * jnp.concatenate([raw_ref[0], raw_ref[1]], axis=-1) + b_ref[...])
    hws_ref[0] = dinv * jnp.dot(h, w_ref[...],
                                preferred_element_type=jnp.float32)


@jax.jit
def _t6(raw, dinv, b2, W3p):
    return pl.pallas_call(
        _t6_body,
        grid=(NC, NMB),
        in_specs=[
            pl.BlockSpec((NC, MB, H // NC), lambda c, m: (0, m, 0)),
            pl.BlockSpec((MB, 1), lambda c, m: (m, 0)),
            pl.BlockSpec((1, H), lambda c, m: (0, 0)),
            pl.BlockSpec((H, H // NC), lambda c, m: (0, 0)),
        ],
        out_specs=pl.BlockSpec((1, MB, H // NC), lambda c, m: (c, m, 0)),
        out_shape=jax.ShapeDtypeStruct((NC, N, H // NC), jnp.float32),
    )(raw, dinv, b2, W3p)


def _t7_body(raw_ref, dinv_ref, b3_ref, out_ref):
    v = dinv_ref[...] * raw_ref[0, :, 0:1] + b3_ref[...]
    out_ref[...] = v


@jax.jit
def _t7(raw3, dinv, b3):
    return pl.pallas_call(
        _t7_body,
        grid=(NMB,),
        in_specs=[
            pl.BlockSpec((1, MB, H // NC), lambda m: (0, m, 0)),
            pl.BlockSpec((MB, 1), lambda m: (m, 0)),
            pl.BlockSpec((1, 1), lambda m: (0, 0)),
        ],
        out_specs=pl.BlockSpec((MB, 1), lambda m: (m, 0)),
        out_shape=jax.ShapeDtypeStruct((N, 1), jnp.float32),
    )(raw3, dinv, b3)


def kernel(x, edge_index, batch, W0, b0, W1, b1, Wfc, bfc, W2, b2, W3, b3):
    src = edge_index[0]
    dst = edge_index[1]
    srcp = jnp.pad(src.reshape(NS, EPT), ((0, 0), (0, EPTP - EPT)))
    src2 = jnp.stack([srcp, srcp + N]).reshape(-1)
    dst_r = jnp.pad(dst.reshape(NS, EPT), ((0, 0), (0, EPTP - EPT)),
                    constant_values=N).reshape(-1)
    onehot = (batch[:, None] == jnp.arange(B)[None, :]).astype(jnp.float32)
    zeros16 = jnp.zeros((N, 128), jnp.float32)
    ones_k = jnp.ones((K, 128), jnp.float32)
    W3p = jnp.pad(W3, ((0, 0), (0, H // NC - 1)))
    b0r = b0.reshape(1, H)
    b1r = b1.reshape(1, H)
    b2r = b2.reshape(1, H)
    bfr = bfc.reshape(1, H)
    b3r = b3.reshape(1, 1)
    W2t = W2[:H]
    W2b = W2[H:]

    deg_raw = _deg_sc(dst_r, zeros16, ones_k)
    dcol = deg_raw[0, :, 0:1]

    hws0, dinv = _t1(x, W0, dcol)
    raw0 = _spmm_sc(hws0.reshape(NC * N, H // NC), src2, dst_r, H // NC)
    hws1, glob = _t2(raw0, dinv, b0r, W1, onehot)
    g2 = _t3(glob, Wfc, bfr, W2b)
    raw1 = _spmm_sc(hws1.reshape(NC * N, H // NC), src2, dst_r, H // NC)
    hws1b = _t4(raw1, dinv, b1r, W1)
    raw1b = _spmm_sc(hws1b.reshape(NC * N, H // NC), src2, dst_r, H // NC)
    hws2 = _t5(raw1b, dinv, b1r, W2t, g2, onehot)
    raw2 = _spmm_sc(hws2.reshape(NC * N, H // NC), src2, dst_r, H // NC)
    hws3 = _t6(raw2, dinv, b2r, W3p)
    raw3 = _spmm_sc(hws3.reshape(NC * N, H // NC), src2, dst_r, H // NC)
    return _t7(raw3, dinv, b3r)
